# Initial kernel scaffold; baseline (speedup 1.0000x reference)
import jax
import jax.numpy as jnp
from jax.experimental import pallas as pl


def kernel(h, edge_index, W, b):
    raise NotImplementedError("write your pallas kernel here")


# speedup vs baseline: 22.2582x; 22.2582x over previous
import functools

import jax
import jax.numpy as jnp
from jax import lax
from jax.experimental import pallas as pl
from jax.experimental.pallas import tpu as pltpu
from jax.experimental.pallas import tpu_sc as plsc

N_NODES = 10000
N_EDGES = 160000
D_FEAT = 256

_NC, _NS, _L = 2, 16, 16
_NW = _NC * _NS
_E_PER_W = N_EDGES // _NW
_CHUNKS = (_E_PER_W + _L - 1) // _L
_E_PAD = _CHUNKS * _L


def _node_proj_body(h_ref, wt_ref, b_ref, out_ref):
    proj = lax.dot_general(
        wt_ref[...], h_ref[...],
        dimension_numbers=(((1,), (1,)), ((), ())),
        preferred_element_type=jnp.float32,
    )
    out_ref[...] = proj + 0.5 * b_ref[...]


def _node_projections(h, W, b):
    wt = W.reshape(2, D_FEAT)
    b2 = b.reshape(1, 1)
    return pl.pallas_call(
        _node_proj_body,
        out_shape=jax.ShapeDtypeStruct((2, N_NODES), jnp.float32),
    )(h, wt, b2)


_sc_mesh = plsc.VectorSubcoreMesh(core_axis_name="c", subcore_axis_name="s")


@functools.partial(
    pl.kernel,
    out_type=jax.ShapeDtypeStruct((N_EDGES,), jnp.float32),
    mesh=_sc_mesh,
    compiler_params=pltpu.CompilerParams(needs_layout_passes=False),
    scratch_types=[
        pltpu.VMEM((N_NODES,), jnp.float32),
        pltpu.VMEM((N_NODES,), jnp.float32),
        pltpu.VMEM((_E_PAD,), jnp.int32),
        pltpu.VMEM((_E_PAD,), jnp.int32),
        pltpu.VMEM((_E_PAD,), jnp.float32),
    ],
)
def _edge_score_kernel(s_hbm, ei_hbm, out_hbm, s1_v, s2_v, src_v, dst_v, o_v):
    wid = lax.axis_index("s") * _NC + lax.axis_index("c")
    base = wid * _E_PER_W
    pltpu.sync_copy(s_hbm.at[0], s1_v)
    pltpu.sync_copy(s_hbm.at[1], s2_v)
    pltpu.sync_copy(ei_hbm.at[pl.ds(base, _E_PER_W)],
                    src_v.at[pl.ds(0, _E_PER_W)])
    pltpu.sync_copy(ei_hbm.at[pl.ds(N_EDGES + base, _E_PER_W)],
                    dst_v.at[pl.ds(0, _E_PER_W)])

    def chunk(i, carry):
        off = pl.multiple_of(i * _L, _L)
        si = jnp.clip(src_v[pl.ds(off, _L)], 0, N_NODES - 1)
        di = jnp.clip(dst_v[pl.ds(off, _L)], 0, N_NODES - 1)
        x = plsc.load_gather(s1_v, [si]) + plsc.load_gather(s2_v, [di])
        o_v[pl.ds(off, _L)] = 1.0 / (1.0 + jnp.exp(-x))
        return carry

    lax.fori_loop(0, _CHUNKS, chunk, 0)
    pltpu.sync_copy(o_v.at[pl.ds(0, _E_PER_W)],
                    out_hbm.at[pl.ds(base, _E_PER_W)])


def kernel(h, edge_index, W, b):
    s = _node_projections(h, W, b)
    ei_flat = edge_index.reshape(2 * N_EDGES)
    return _edge_score_kernel(s, ei_flat)


# speedup vs baseline: 30.6577x; 1.3774x over previous
import functools

import jax
import jax.numpy as jnp
from jax import lax
from jax.experimental import pallas as pl
from jax.experimental.pallas import tpu as pltpu
from jax.experimental.pallas import tpu_sc as plsc

N_NODES = 10000
N_EDGES = 160000
D_FEAT = 256

_NC, _NS, _L = 2, 16, 16
_NW = _NC * _NS
_CT = 128
_MAIN_E = 39 * _CT
_MAIN_CHUNKS = _MAIN_E // _L
_TAIL_E0 = _NW * _MAIN_E
_TAIL_CHUNKS = _CT // _L
_BUF_E = _MAIN_E + _CT


def _node_proj_body(h_ref, wt_ref, b_ref, out_ref):
    proj = lax.dot_general(
        wt_ref[...], h_ref[...],
        dimension_numbers=(((1,), (1,)), ((), ())),
        preferred_element_type=jnp.float32,
    )
    out_ref[...] = proj + 0.5 * b_ref[...]


def _node_projections(h, W, b):
    wt = W.reshape(2, D_FEAT)
    b2 = b.reshape(1, 1)
    return pl.pallas_call(
        _node_proj_body,
        out_shape=jax.ShapeDtypeStruct((2, N_NODES), jnp.float32),
    )(h, wt, b2)


_sc_mesh = plsc.VectorSubcoreMesh(core_axis_name="c", subcore_axis_name="s")


@functools.partial(
    pl.kernel,
    out_type=jax.ShapeDtypeStruct((N_EDGES,), jnp.float32),
    mesh=_sc_mesh,
    compiler_params=pltpu.CompilerParams(needs_layout_passes=False),
    scratch_types=[
        pltpu.VMEM((N_NODES,), jnp.float32),
        pltpu.VMEM((N_NODES,), jnp.float32),
        pltpu.VMEM((2, _BUF_E), jnp.int32),
        pltpu.VMEM((_BUF_E,), jnp.float32),
        pltpu.SemaphoreType.DMA,
    ],
)
def _edge_score_kernel(s_hbm, ei_hbm, out_hbm, s1_v, s2_v, ei_v, o_v, sem):
    wid = lax.axis_index("s") * _NC + lax.axis_index("c")
    base = wid * _MAIN_E
    has_tail = wid < 2

    d1 = pltpu.async_copy(s_hbm.at[0], s1_v, sem)
    d2 = pltpu.async_copy(s_hbm.at[1], s2_v, sem)
    d3 = pltpu.async_copy(ei_hbm.at[:, pl.ds(base, _MAIN_E)],
                          ei_v.at[:, pl.ds(0, _MAIN_E)], sem)

    @pl.when(has_tail)
    def _tail_idx():
        pltpu.async_copy(ei_hbm.at[:, pl.ds(_TAIL_E0 + wid * _CT, _CT)],
                         ei_v.at[:, pl.ds(_MAIN_E, _CT)], sem).wait()

    d1.wait()
    d2.wait()
    d3.wait()

    def score_chunk(off):
        si = ei_v[0, pl.ds(off, _L)]
        di = ei_v[1, pl.ds(off, _L)]
        x = plsc.load_gather(s1_v, [si]) + plsc.load_gather(s2_v, [di])
        o_v[pl.ds(off, _L)] = 1.0 / (1.0 + jnp.exp(-x))

    @plsc.parallel_loop(0, _MAIN_CHUNKS, unroll=8)
    def _main(i):
        score_chunk(pl.multiple_of(i * _L, _L))

    @pl.when(has_tail)
    def _tail():
        @plsc.parallel_loop(0, _TAIL_CHUNKS, unroll=8)
        def _t(i):
            score_chunk(pl.multiple_of(_MAIN_E + i * _L, _L))

    pltpu.sync_copy(o_v.at[pl.ds(0, _MAIN_E)],
                    out_hbm.at[pl.ds(base, _MAIN_E)])

    @pl.when(has_tail)
    def _tail_out():
        pltpu.sync_copy(o_v.at[pl.ds(_MAIN_E, _CT)],
                        out_hbm.at[pl.ds(_TAIL_E0 + wid * _CT, _CT)])


def kernel(h, edge_index, W, b):
    s = _node_projections(h, W, b)
    return _edge_score_kernel(s, edge_index)
